# baseline (speedup 1.0000x reference)
import functools

import jax
import jax.numpy as jnp
from jax import lax
from jax.experimental import pallas as pl
from jax.experimental.pallas import tpu as pltpu
from jax.experimental.pallas import tpu_sc as plsc

E = 3_200_000
EDGE_DIM = 16
GLOBAL_DIM = 128
NUM_GRAPHS = 1024

NC, NS, L = 2, 16, 16
NW = NC * NS
TCOLS = E // 128
CPW = 780
BCOLS = 10
NBLK = CPW // BCOLS
BE = BCOLS * 128
GPB = BE // L
PITCH = EDGE_DIM + 1
EXTRA0 = NW * CPW
TBL = NUM_GRAPHS * L

_GATHER_DNUMS = lax.GatherDimensionNumbers(
    offset_dims=(), collapsed_slice_dims=(0,), start_index_map=(0,))


def _bcast_lane(v, r):
    idx = jnp.full((L, 1), r, jnp.int32)
    return lax.gather(v, idx, _GATHER_DNUMS, slice_sizes=(1,),
                      mode=lax.GatherScatterMode.PROMISE_IN_BOUNDS)


_MESH = plsc.VectorSubcoreMesh(
    core_axis_name="c", subcore_axis_name="s", num_cores=NC, num_subcores=NS
)


@functools.partial(
    pl.kernel,
    out_type=(
        jax.ShapeDtypeStruct((NUM_GRAPHS, NW * EDGE_DIM), jnp.float32),
        jax.ShapeDtypeStruct((NUM_GRAPHS, NW * EDGE_DIM), jnp.float32),
    ),
    mesh=_MESH,
    compiler_params=pltpu.CompilerParams(
        needs_layout_passes=False, use_tc_tiling_on_sc=False),
    scratch_types=[
        pltpu.VMEM((NUM_GRAPHS, EDGE_DIM), jnp.float32),
        pltpu.VMEM((NUM_GRAPHS, EDGE_DIM), jnp.float32),
        pltpu.VMEM((2 * BCOLS, 8, 128), jnp.float32),
        pltpu.VMEM((2 * BCOLS, 8, 128), jnp.float32),
        pltpu.VMEM((BE * PITCH,), jnp.float32),
        pltpu.VMEM((BE + L,), jnp.int32),
        pltpu.VMEM((BE + L,), jnp.int32),
        pltpu.SemaphoreType.DMA,
        pltpu.SemaphoreType.DMA,
    ],
)
def _sc_segment_sums(edges_hbm, ids_hbm, sums_hbm, counts_hbm,
                     sums_v, counts_v, raw_a, raw_b, packed_v, idb_a, idb_b,
                     sem_a, sem_b):
    wid = lax.axis_index("s") * NC + lax.axis_index("c")
    zeros = jnp.zeros((L,), jnp.float32)
    ones = jnp.ones((L,), jnp.float32)
    iota = lax.iota(jnp.int32, L)
    p17 = iota * PITCH
    col0_w = wid * CPW

    def zero_body(i, carry):
        sums_v[i] = zeros
        counts_v[i] = zeros
        return carry

    lax.fori_loop(0, NUM_GRAPHS, zero_body, 0)

    def start_copies(col0, raw, idb, sem):
        pltpu.async_copy(edges_hbm.at[0, pl.ds(col0, BCOLS)],
                         raw.at[pl.ds(0, BCOLS)], sem)
        pltpu.async_copy(edges_hbm.at[1, pl.ds(col0, BCOLS)],
                         raw.at[pl.ds(BCOLS, BCOLS)], sem)
        pltpu.async_copy(ids_hbm.at[pl.ds(col0 * 128, BE)],
                         idb.at[pl.ds(0, BE)], sem)

    def wait_copies(raw, idb, sem):
        pltpu.make_async_copy(edges_hbm.at[0, pl.ds(0, BCOLS)],
                              raw.at[pl.ds(0, BCOLS)], sem).wait()
        pltpu.make_async_copy(edges_hbm.at[1, pl.ds(0, BCOLS)],
                              raw.at[pl.ds(BCOLS, BCOLS)], sem).wait()
        pltpu.make_async_copy(ids_hbm.at[pl.ds(0, BE)],
                              idb.at[pl.ds(0, BE)], sem).wait()

    RP = 8

    def repack(raw):
        def tc_body(tc, carry):
            pend = list(carry)
            e17 = tc * (128 * PITCH)
            for h in range(2):
                for fr in range(8):
                    f = h * 8 + fr
                    for cs in range(8):
                        v = raw[h * BCOLS + tc, fr, pl.ds(cs * 16, L)]
                        sb = e17 + cs * (16 * PITCH) + f
                        pend.append((v, sb))
                        vv, sbo = pend.pop(0)
                        plsc.store_scatter(packed_v, [p17 + sbo], vv)
            return tuple(pend)

        init = tuple((zeros, jnp.int32(0)) for _ in range(RP))
        fin = lax.fori_loop(0, BCOLS, tc_body, init)
        for vv, sbo in fin:
            plsc.store_scatter(packed_v, [p17 + sbo], vv)

    PIPE = 6

    def main_pass(idb):
        def grp_body(g, carry):
            ids_g, pend = carry
            pend_rows, pend_idxs = list(pend[0]), list(pend[1])
            ids_next = idb[pl.ds((g + 1) * L, L)]
            plsc.addupdate_scatter(counts_v, [ids_g, iota], ones)
            base = g * (L * PITCH)
            for r in range(L):
                pend_rows.append(packed_v[pl.ds(base + r * PITCH, L)])
                pend_idxs.append(_bcast_lane(ids_g, r))
                plsc.addupdate_scatter(
                    sums_v, [pend_idxs.pop(0), iota], pend_rows.pop(0))
            return (ids_next, (tuple(pend_rows), tuple(pend_idxs)))

        ids_0 = idb[pl.ds(0, L)]
        init = (ids_0,
                (tuple(jnp.zeros((L,), jnp.float32) for _ in range(PIPE)),
                 (iota,) * PIPE))
        _, (fin_rows, fin_idxs) = lax.fori_loop(0, GPB, grp_body, init, unroll=4)
        for k in range(PIPE):
            plsc.addupdate_scatter(sums_v, [fin_idxs[k], iota], fin_rows[k])

    start_copies(col0_w, raw_a, idb_a, sem_a)

    def pair_body(p, carry):
        b0 = 2 * p
        start_copies(col0_w + (b0 + 1) * BCOLS, raw_b, idb_b, sem_b)
        wait_copies(raw_a, idb_a, sem_a)
        repack(raw_a)
        main_pass(idb_a)

        @pl.when(b0 + 2 < NBLK)
        def _():
            start_copies(col0_w + (b0 + 2) * BCOLS, raw_a, idb_a, sem_a)

        wait_copies(raw_b, idb_b, sem_b)
        repack(raw_b)
        main_pass(idb_b)
        return carry

    lax.fori_loop(0, NBLK // 2, pair_body, 0)

    @pl.when(wid < (TCOLS - EXTRA0) // BCOLS)
    def _():
        colx = EXTRA0 + wid * BCOLS
        start_copies(colx, raw_a, idb_a, sem_a)
        wait_copies(raw_a, idb_a, sem_a)
        repack(raw_a)
        main_pass(idb_a)

    pltpu.sync_copy(sums_v, sums_hbm.at[:, pl.ds(wid * EDGE_DIM, EDGE_DIM)])
    pltpu.sync_copy(counts_v,
                    counts_hbm.at[:, pl.ds(wid * EDGE_DIM, EDGE_DIM)])


def _tc_body(sums_ref, counts_ref, W_ref, b_ref, gamma_ref, beta_ref, out_ref):
    s = sums_ref[...]
    total = s[:, 0:EDGE_DIM]
    for k in range(1, NW):
        total = total + s[:, k * EDGE_DIM:(k + 1) * EDGE_DIM]
    cnt = jnp.sum(counts_ref[...], axis=1, keepdims=True)
    u = total / jnp.maximum(cnt, 1.0)
    u = jnp.dot(u, W_ref[...], preferred_element_type=jnp.float32)
    u = u + b_ref[...][None, :]
    mean = jnp.mean(u, axis=1, keepdims=True)
    var = jnp.mean((u - mean) ** 2, axis=1, keepdims=True)
    normed = (u - mean) * lax.rsqrt(var + 1e-5)
    out_ref[...] = normed * gamma_ref[...][None, :] + beta_ref[...][None, :]


_tc_finish = pl.pallas_call(
    _tc_body,
    out_shape=jax.ShapeDtypeStruct((NUM_GRAPHS, GLOBAL_DIM), jnp.float32),
)


def kernel(edge_attr, batch, W, b, gamma, beta):
    ids = batch.astype(jnp.int32)
    edges4 = edge_attr.reshape(TCOLS, 128, 2, 8).transpose(2, 0, 3, 1)
    sums, counts = _sc_segment_sums(edges4, ids)
    return _tc_finish(sums, counts, W, b, gamma, beta)


# speedup vs baseline: 1.1154x; 1.1154x over previous
import functools

import jax
import jax.numpy as jnp
from jax import lax
from jax.experimental import pallas as pl
from jax.experimental.pallas import tpu as pltpu
from jax.experimental.pallas import tpu_sc as plsc

E = 3_200_000
EDGE_DIM = 16
GLOBAL_DIM = 128
NUM_GRAPHS = 1024

NC, NS, L = 2, 16, 16
NW = NC * NS
TCOLS = E // 128
CPW = 780
BCOLS = 10
NBLK = CPW // BCOLS
BE = BCOLS * 128
GPB = BE // L
PITCH = EDGE_DIM + 1
EXTRA0 = NW * CPW
TBL = NUM_GRAPHS * L

_GATHER_DNUMS = lax.GatherDimensionNumbers(
    offset_dims=(), collapsed_slice_dims=(0,), start_index_map=(0,))


def _bcast_lane(v, r):
    idx = jnp.full((L, 1), r, jnp.int32)
    return lax.gather(v, idx, _GATHER_DNUMS, slice_sizes=(1,),
                      mode=lax.GatherScatterMode.PROMISE_IN_BOUNDS)


_MESH = plsc.VectorSubcoreMesh(
    core_axis_name="c", subcore_axis_name="s", num_cores=NC, num_subcores=NS
)


@functools.partial(
    pl.kernel,
    out_type=(
        jax.ShapeDtypeStruct((NUM_GRAPHS, NW * EDGE_DIM), jnp.float32),
        jax.ShapeDtypeStruct((NUM_GRAPHS, NW * EDGE_DIM), jnp.float32),
    ),
    mesh=_MESH,
    compiler_params=pltpu.CompilerParams(
        needs_layout_passes=False, use_tc_tiling_on_sc=False),
    scratch_types=[
        pltpu.VMEM((NUM_GRAPHS, EDGE_DIM), jnp.float32),
        pltpu.VMEM((NUM_GRAPHS, EDGE_DIM), jnp.float32),
        pltpu.VMEM((2 * BCOLS, 8, 128), jnp.float32),
        pltpu.VMEM((2 * BCOLS, 8, 128), jnp.float32),
        pltpu.VMEM((BE * PITCH,), jnp.float32),
        pltpu.VMEM((BE + L,), jnp.int32),
        pltpu.VMEM((BE + L,), jnp.int32),
        pltpu.SemaphoreType.DMA,
        pltpu.SemaphoreType.DMA,
    ],
)
def _sc_segment_sums(edges_hbm, ids_hbm, sums_hbm, counts_hbm,
                     sums_v, counts_v, raw_a, raw_b, packed_v, idb_a, idb_b,
                     sem_a, sem_b):
    wid = lax.axis_index("s") * NC + lax.axis_index("c")
    zeros = jnp.zeros((L,), jnp.float32)
    ones = jnp.ones((L,), jnp.float32)
    iota = lax.iota(jnp.int32, L)
    p17 = iota * PITCH
    col0_w = wid * CPW

    def zero_body(i, carry):
        sums_v[i] = zeros
        counts_v[i] = zeros
        return carry

    lax.fori_loop(0, NUM_GRAPHS, zero_body, 0)

    def start_copies(col0, raw, idb, sem):
        pltpu.async_copy(edges_hbm.at[0, pl.ds(col0, BCOLS)],
                         raw.at[pl.ds(0, BCOLS)], sem)
        pltpu.async_copy(edges_hbm.at[1, pl.ds(col0, BCOLS)],
                         raw.at[pl.ds(BCOLS, BCOLS)], sem)
        pltpu.async_copy(ids_hbm.at[pl.ds(col0 * 128, BE)],
                         idb.at[pl.ds(0, BE)], sem)

    def wait_copies(raw, idb, sem):
        pltpu.make_async_copy(edges_hbm.at[0, pl.ds(0, BCOLS)],
                              raw.at[pl.ds(0, BCOLS)], sem).wait()
        pltpu.make_async_copy(edges_hbm.at[1, pl.ds(0, BCOLS)],
                              raw.at[pl.ds(BCOLS, BCOLS)], sem).wait()
        pltpu.make_async_copy(ids_hbm.at[pl.ds(0, BE)],
                              idb.at[pl.ds(0, BE)], sem).wait()

    RP = 12

    def repack(raw):
        def tc_body(tc, carry):
            pend = list(carry)
            e17 = tc * (128 * PITCH)
            for h in range(2):
                for fr in range(8):
                    f = h * 8 + fr
                    for cs in range(8):
                        v = raw[h * BCOLS + tc, fr, pl.ds(cs * 16, L)]
                        sb = e17 + cs * (16 * PITCH) + f
                        pend.append((v, sb))
                        vv, sbo = pend.pop(0)
                        plsc.store_scatter(packed_v, [p17 + sbo], vv)
            return tuple(pend)

        init = tuple((zeros, jnp.int32(0)) for _ in range(RP))
        fin = lax.fori_loop(0, BCOLS, tc_body, init)
        for vv, sbo in fin:
            plsc.store_scatter(packed_v, [p17 + sbo], vv)

    PIPE = 4

    def main_pass(idb):
        def grp_body(g, carry):
            ids_g, pend = carry
            pend_rows, pend_idxs = list(pend[0]), list(pend[1])
            ids_next = idb[pl.ds((g + 1) * L, L)]
            plsc.addupdate_scatter(counts_v, [ids_g, iota], ones)
            base = g * (L * PITCH)
            for r in range(L):
                pend_rows.append(packed_v[pl.ds(base + r * PITCH, L)])
                pend_idxs.append(_bcast_lane(ids_g, r))
                plsc.addupdate_scatter(
                    sums_v, [pend_idxs.pop(0), iota], pend_rows.pop(0))
            return (ids_next, (tuple(pend_rows), tuple(pend_idxs)))

        ids_0 = idb[pl.ds(0, L)]
        init = (ids_0,
                (tuple(jnp.zeros((L,), jnp.float32) for _ in range(PIPE)),
                 (iota,) * PIPE))
        _, (fin_rows, fin_idxs) = lax.fori_loop(0, GPB, grp_body, init, unroll=2)
        for k in range(PIPE):
            plsc.addupdate_scatter(sums_v, [fin_idxs[k], iota], fin_rows[k])

    start_copies(col0_w, raw_a, idb_a, sem_a)

    def pair_body(p, carry):
        b0 = 2 * p
        start_copies(col0_w + (b0 + 1) * BCOLS, raw_b, idb_b, sem_b)
        wait_copies(raw_a, idb_a, sem_a)
        repack(raw_a)
        main_pass(idb_a)

        @pl.when(b0 + 2 < NBLK)
        def _():
            start_copies(col0_w + (b0 + 2) * BCOLS, raw_a, idb_a, sem_a)

        wait_copies(raw_b, idb_b, sem_b)
        repack(raw_b)
        main_pass(idb_b)
        return carry

    lax.fori_loop(0, NBLK // 2, pair_body, 0)

    @pl.when(wid < (TCOLS - EXTRA0) // BCOLS)
    def _():
        colx = EXTRA0 + wid * BCOLS
        start_copies(colx, raw_a, idb_a, sem_a)
        wait_copies(raw_a, idb_a, sem_a)
        repack(raw_a)
        main_pass(idb_a)

    pltpu.sync_copy(sums_v, sums_hbm.at[:, pl.ds(wid * EDGE_DIM, EDGE_DIM)])
    pltpu.sync_copy(counts_v,
                    counts_hbm.at[:, pl.ds(wid * EDGE_DIM, EDGE_DIM)])


def _tc_body(sums_ref, counts_ref, W_ref, b_ref, gamma_ref, beta_ref, out_ref):
    sel = (jnp.arange(NW * EDGE_DIM, dtype=jnp.int32)[:, None] % EDGE_DIM
           == jnp.arange(EDGE_DIM, dtype=jnp.int32)[None, :]
           ).astype(jnp.float32)
    total = jnp.dot(sums_ref[...], sel,
                    preferred_element_type=jnp.float32)
    cnt = jnp.sum(counts_ref[...], axis=1, keepdims=True)
    u = total / jnp.maximum(cnt, 1.0)
    u = jnp.dot(u, W_ref[...], preferred_element_type=jnp.float32)
    u = u + b_ref[...][None, :]
    mean = jnp.mean(u, axis=1, keepdims=True)
    var = jnp.mean((u - mean) ** 2, axis=1, keepdims=True)
    normed = (u - mean) * lax.rsqrt(var + 1e-5)
    out_ref[...] = normed * gamma_ref[...][None, :] + beta_ref[...][None, :]


_tc_finish = pl.pallas_call(
    _tc_body,
    out_shape=jax.ShapeDtypeStruct((NUM_GRAPHS, GLOBAL_DIM), jnp.float32),
)


def kernel(edge_attr, batch, W, b, gamma, beta):
    ids = batch.astype(jnp.int32)
    edges4 = edge_attr.reshape(TCOLS, 128, 2, 8).transpose(2, 0, 3, 1)
    sums, counts = _sc_segment_sums(edges4, ids)
    return _tc_finish(sums, counts, W, b, gamma, beta)


# speedup vs baseline: 1.1219x; 1.0059x over previous
import functools

import jax
import jax.numpy as jnp
from jax import lax
from jax.experimental import pallas as pl
from jax.experimental.pallas import tpu as pltpu
from jax.experimental.pallas import tpu_sc as plsc

E = 3_200_000
EDGE_DIM = 16
GLOBAL_DIM = 128
NUM_GRAPHS = 1024

NC, NS, L = 2, 16, 16
NW = NC * NS
TCOLS = E // 128
CPW = 780
BCOLS = 10
NBLK = CPW // BCOLS
BE = BCOLS * 128
GPB = BE // L
PITCH = EDGE_DIM + 1
EXTRA0 = NW * CPW
TBL = NUM_GRAPHS * L

_GATHER_DNUMS = lax.GatherDimensionNumbers(
    offset_dims=(), collapsed_slice_dims=(0,), start_index_map=(0,))


def _bcast_lane(v, r):
    idx = jnp.full((L, 1), r, jnp.int32)
    return lax.gather(v, idx, _GATHER_DNUMS, slice_sizes=(1,),
                      mode=lax.GatherScatterMode.PROMISE_IN_BOUNDS)


_MESH = plsc.VectorSubcoreMesh(
    core_axis_name="c", subcore_axis_name="s", num_cores=NC, num_subcores=NS
)


@functools.partial(
    pl.kernel,
    out_type=(
        jax.ShapeDtypeStruct((NUM_GRAPHS, NW * EDGE_DIM), jnp.float32),
        jax.ShapeDtypeStruct((NUM_GRAPHS, NW * EDGE_DIM), jnp.float32),
    ),
    mesh=_MESH,
    compiler_params=pltpu.CompilerParams(
        needs_layout_passes=False, use_tc_tiling_on_sc=False),
    scratch_types=[
        pltpu.VMEM((NUM_GRAPHS, EDGE_DIM), jnp.float32),
        pltpu.VMEM((NUM_GRAPHS, EDGE_DIM), jnp.float32),
        pltpu.VMEM((2 * BCOLS, 8, 128), jnp.float32),
        pltpu.VMEM((2 * BCOLS, 8, 128), jnp.float32),
        pltpu.VMEM((BE * PITCH,), jnp.float32),
        pltpu.VMEM((BE + L,), jnp.int32),
        pltpu.VMEM((BE + L,), jnp.int32),
        pltpu.SemaphoreType.DMA,
        pltpu.SemaphoreType.DMA,
    ],
)
def _sc_segment_sums(edges_hbm, ids_hbm, sums_hbm, counts_hbm,
                     sums_v, counts_v, raw_a, raw_b, packed_v, idb_a, idb_b,
                     sem_a, sem_b):
    wid = lax.axis_index("s") * NC + lax.axis_index("c")
    zeros = jnp.zeros((L,), jnp.float32)
    ones = jnp.ones((L,), jnp.float32)
    iota = lax.iota(jnp.int32, L)
    p17 = iota * PITCH
    col0_w = wid * CPW

    def zero_body(i, carry):
        sums_v[i] = zeros
        counts_v[i] = zeros
        return carry

    lax.fori_loop(0, NUM_GRAPHS, zero_body, 0)

    def start_copies(col0, raw, idb, sem):
        pltpu.async_copy(edges_hbm.at[0, pl.ds(col0, BCOLS)],
                         raw.at[pl.ds(0, BCOLS)], sem)
        pltpu.async_copy(edges_hbm.at[1, pl.ds(col0, BCOLS)],
                         raw.at[pl.ds(BCOLS, BCOLS)], sem)
        pltpu.async_copy(ids_hbm.at[pl.ds(col0 * 128, BE)],
                         idb.at[pl.ds(0, BE)], sem)

    def wait_copies(raw, idb, sem):
        pltpu.make_async_copy(edges_hbm.at[0, pl.ds(0, BCOLS)],
                              raw.at[pl.ds(0, BCOLS)], sem).wait()
        pltpu.make_async_copy(edges_hbm.at[1, pl.ds(0, BCOLS)],
                              raw.at[pl.ds(BCOLS, BCOLS)], sem).wait()
        pltpu.make_async_copy(ids_hbm.at[pl.ds(0, BE)],
                              idb.at[pl.ds(0, BE)], sem).wait()

    RP = 16

    def repack(raw):
        def tc_body(tc, carry):
            pend = list(carry)
            e17 = tc * (128 * PITCH)
            for h in range(2):
                for fr in range(8):
                    f = h * 8 + fr
                    for cs in range(8):
                        v = raw[h * BCOLS + tc, fr, pl.ds(cs * 16, L)]
                        sb = e17 + cs * (16 * PITCH) + f
                        pend.append((v, sb))
                        vv, sbo = pend.pop(0)
                        plsc.store_scatter(packed_v, [p17 + sbo], vv)
            return tuple(pend)

        init = tuple((zeros, jnp.int32(0)) for _ in range(RP))
        fin = lax.fori_loop(0, BCOLS, tc_body, init)
        for vv, sbo in fin:
            plsc.store_scatter(packed_v, [p17 + sbo], vv)

    PIPE = 6

    def main_pass(idb):
        def grp_body(g, carry):
            ids_g, pend = carry
            pend_rows, pend_idxs = list(pend[0]), list(pend[1])
            ids_next = idb[pl.ds((g + 1) * L, L)]
            plsc.addupdate_scatter(counts_v, [ids_g, iota], ones)
            base = g * (L * PITCH)
            for r in range(L):
                pend_rows.append(packed_v[pl.ds(base + r * PITCH, L)])
                pend_idxs.append(_bcast_lane(ids_g, r))
                plsc.addupdate_scatter(
                    sums_v, [pend_idxs.pop(0), iota], pend_rows.pop(0))
            return (ids_next, (tuple(pend_rows), tuple(pend_idxs)))

        ids_0 = idb[pl.ds(0, L)]
        init = (ids_0,
                (tuple(jnp.zeros((L,), jnp.float32) for _ in range(PIPE)),
                 (iota,) * PIPE))
        _, (fin_rows, fin_idxs) = lax.fori_loop(0, GPB, grp_body, init, unroll=2)
        for k in range(PIPE):
            plsc.addupdate_scatter(sums_v, [fin_idxs[k], iota], fin_rows[k])

    start_copies(col0_w, raw_a, idb_a, sem_a)

    def pair_body(p, carry):
        b0 = 2 * p
        start_copies(col0_w + (b0 + 1) * BCOLS, raw_b, idb_b, sem_b)
        wait_copies(raw_a, idb_a, sem_a)
        repack(raw_a)
        main_pass(idb_a)

        @pl.when(b0 + 2 < NBLK)
        def _():
            start_copies(col0_w + (b0 + 2) * BCOLS, raw_a, idb_a, sem_a)

        wait_copies(raw_b, idb_b, sem_b)
        repack(raw_b)
        main_pass(idb_b)
        return carry

    lax.fori_loop(0, NBLK // 2, pair_body, 0)

    @pl.when(wid < (TCOLS - EXTRA0) // BCOLS)
    def _():
        colx = EXTRA0 + wid * BCOLS
        start_copies(colx, raw_a, idb_a, sem_a)
        wait_copies(raw_a, idb_a, sem_a)
        repack(raw_a)
        main_pass(idb_a)

    pltpu.sync_copy(sums_v, sums_hbm.at[:, pl.ds(wid * EDGE_DIM, EDGE_DIM)])
    pltpu.sync_copy(counts_v,
                    counts_hbm.at[:, pl.ds(wid * EDGE_DIM, EDGE_DIM)])


def _tc_body(sums_ref, counts_ref, W_ref, b_ref, gamma_ref, beta_ref, out_ref):
    sel = (jnp.arange(NW * EDGE_DIM, dtype=jnp.int32)[:, None] % EDGE_DIM
           == jnp.arange(EDGE_DIM, dtype=jnp.int32)[None, :]
           ).astype(jnp.float32)
    total = jnp.dot(sums_ref[...], sel,
                    preferred_element_type=jnp.float32)
    cnt = jnp.sum(counts_ref[...], axis=1, keepdims=True)
    u = total / jnp.maximum(cnt, 1.0)
    u = jnp.dot(u, W_ref[...], preferred_element_type=jnp.float32)
    u = u + b_ref[...][None, :]
    mean = jnp.mean(u, axis=1, keepdims=True)
    var = jnp.mean((u - mean) ** 2, axis=1, keepdims=True)
    normed = (u - mean) * lax.rsqrt(var + 1e-5)
    out_ref[...] = normed * gamma_ref[...][None, :] + beta_ref[...][None, :]


_tc_finish = pl.pallas_call(
    _tc_body,
    out_shape=jax.ShapeDtypeStruct((NUM_GRAPHS, GLOBAL_DIM), jnp.float32),
)


def kernel(edge_attr, batch, W, b, gamma, beta):
    ids = batch.astype(jnp.int32)
    edges4 = edge_attr.reshape(TCOLS, 128, 2, 8).transpose(2, 0, 3, 1)
    sums, counts = _sc_segment_sums(edges4, ids)
    return _tc_finish(sums, counts, W, b, gamma, beta)
